# Initial kernel scaffold; baseline (speedup 1.0000x reference)
import jax
import jax.numpy as jnp
from jax.experimental import pallas as pl


def kernel(x, W, b):
    raise NotImplementedError("write your pallas kernel here")


# speedup vs baseline: 2.9527x; 2.9527x over previous
import jax
import jax.numpy as jnp
from jax.experimental import pallas as pl
from jax.experimental.pallas import tpu as pltpu

_BATCH = 128
_DIM = 4096
_TEMP = 2.0
_BK = 512
_NBLK = _DIM // _BK


def _pass1(x_ref, b_ref, w_ref, gx_ref):
    i = pl.program_id(0)

    @pl.when(i == 0)
    def _init():
        gx_ref[...] = jnp.broadcast_to(b_ref[...], (_BATCH, _DIM))

    w = w_ref[...]
    xi = x_ref[:, pl.ds(i * _BK, _BK)]
    gx_ref[...] += jnp.dot(xi, w, preferred_element_type=jnp.float32)
    colpart = jax.lax.dot_general(
        x_ref[...], w, (((1,), (1,)), ((), ())),
        preferred_element_type=jnp.float32)
    gx_ref[:, pl.ds(i * _BK, _BK)] += colpart


def _pass2(x_ref, gx_ref, g_ref, u_ref, w_ref, out_ref, c_ref, r_ref):
    i = pl.program_id(0)

    @pl.when(i == 0)
    def _sample():
        x = x_ref[...]
        logits = gx_ref[...] * ((1.0 - 2.0 * x) / _TEMP)
        z = logits + g_ref[...]
        m = jnp.max(z, axis=1, keepdims=True)
        iota = jax.lax.broadcasted_iota(jnp.int32, (_BATCH, _DIM), 1)
        idx = jnp.min(jnp.where(z >= m, iota, _DIM), axis=1, keepdims=True)
        c_ref[...] = (iota == idx).astype(jnp.float32)
        r_ref[...] = jnp.zeros_like(r_ref)

    w = w_ref[...]
    ci = c_ref[:, pl.ds(i * _BK, _BK)]
    r_ref[...] += jnp.dot(ci, w, preferred_element_type=jnp.float32)
    r_ref[:, pl.ds(i * _BK, _BK)] += jax.lax.dot_general(
        c_ref[...], w, (((1,), (1,)), ((), ())),
        preferred_element_type=jnp.float32)

    @pl.when(i == _NBLK - 1)
    def _accept():
        x = x_ref[...]
        gx = gx_ref[...]
        c = c_ref[...]
        r = r_ref[...]
        logits = gx * ((1.0 - 2.0 * x) / _TEMP)
        m = jnp.max(logits, axis=1, keepdims=True)
        lse = m + jnp.log(jnp.sum(jnp.exp(logits - m), axis=1, keepdims=True))
        lp_fwd = jnp.sum(c * logits, axis=1, keepdims=True) - lse

        s = 1.0 - 2.0 * jnp.sum(c * x, axis=1, keepdims=True)
        x_delta = x + s * c
        rev_logits = (gx + s * r) * ((1.0 - 2.0 * x_delta) / _TEMP)
        m2 = jnp.max(rev_logits, axis=1, keepdims=True)
        lse2 = m2 + jnp.log(
            jnp.sum(jnp.exp(rev_logits - m2), axis=1, keepdims=True))
        lp_rev = jnp.sum(c * rev_logits, axis=1, keepdims=True) - lse2

        gx_at = jnp.sum(c * gx, axis=1, keepdims=True)
        diag = 0.5 * jnp.sum(c * r, axis=1, keepdims=True)
        m_term = s * gx_at + diag
        la = m_term + lp_rev - lp_fwd
        a = (jnp.exp(la) > u_ref[...]).astype(jnp.float32)
        out_ref[...] = x + (a * s) * c


def kernel(x, W, b):
    key = jax.random.key(42)
    k1, k2 = jax.random.split(key)
    g = jax.random.gumbel(k1, (_BATCH, _DIM), jnp.float32)
    u = jax.random.uniform(k2, (_BATCH,), jnp.float32).reshape(_BATCH, 1)
    b2 = b.reshape(1, _DIM)

    full = pl.BlockSpec((_BATCH, _DIM), lambda i: (0, 0))
    wspec = pl.BlockSpec((_BK, _DIM), lambda i: (i, 0))
    params = pltpu.CompilerParams(dimension_semantics=("arbitrary",))

    gx = pl.pallas_call(
        _pass1,
        grid=(_NBLK,),
        in_specs=[full, pl.BlockSpec((1, _DIM), lambda i: (0, 0)), wspec],
        out_specs=full,
        out_shape=jax.ShapeDtypeStruct((_BATCH, _DIM), jnp.float32),
        compiler_params=params,
    )(x, b2, W)

    out = pl.pallas_call(
        _pass2,
        grid=(_NBLK,),
        in_specs=[full, full, full,
                  pl.BlockSpec((_BATCH, 1), lambda i: (0, 0)), wspec],
        out_specs=full,
        out_shape=jax.ShapeDtypeStruct((_BATCH, _DIM), jnp.float32),
        scratch_shapes=[pltpu.VMEM((_BATCH, _DIM), jnp.float32),
                        pltpu.VMEM((_BATCH, _DIM), jnp.float32)],
        compiler_params=params,
    )(x, gx, g, u, W)
    return out
